# Initial kernel scaffold; baseline (speedup 1.0000x reference)
import jax
import jax.numpy as jnp
from jax.experimental import pallas as pl


def kernel(features, motion, W1, b1, g1, be1, W2, b2, g2, be2, W3, b3, g3, be3, W4, b4, g4, be4, delta):
    raise NotImplementedError("write your pallas kernel here")


# speedup vs baseline: 13.3420x; 13.3420x over previous
import functools

import jax
import jax.numpy as jnp
from jax import lax
from jax.experimental import pallas as pl
from jax.experimental.pallas import tpu as pltpu
from jax.experimental.pallas import tpu_sc as plsc

B, C, N = 8, 128, 2048
K = 9
TN = 256
NBLK = N // TN
TR = 512
NB = N // TR
NSLOT = 8
TOT = 2 * B * N * K
NW = 32
RPW = TOT // NW
CHUNK = 128
NCH = RPW // CHUNK
CNT1 = B * N * 3
CNT2 = B * N
EPS = 1e-5
NEG = -3.0e38
BIGI = 1 << 30


def _topk_body(cdim, slot_base, xt_ref, x_ref, o_ref):
    b = pl.program_id(0)
    xb = xt_ref[0]
    xf = x_ref[0]
    xx = jnp.sum(xf * xf, axis=0, keepdims=True)
    s = jnp.dot(xb, xf, preferred_element_type=jnp.float32)
    v = 2.0 * s - xx
    iota_m = lax.broadcasted_iota(jnp.int32, (TN, N), 1)
    iota16 = lax.broadcasted_iota(jnp.int32, (TN, 16), 1)
    acc = jnp.zeros((TN, 16), jnp.int32)
    gbase = b * (NSLOT * N)
    for r in range(K):
        mx = jnp.max(v, axis=1, keepdims=True)
        hit = v >= mx
        idxr = jnp.min(jnp.where(hit, iota_m, BIGI), axis=1, keepdims=True)
        slot = slot_base + (r % 3)
        gidx = idxr + (gbase + slot * N)
        acc = jnp.where(iota16 == r, jnp.broadcast_to(gidx, (TN, 16)), acc)
        v = jnp.where(iota_m == idxr, NEG, v)
    o_ref[0] = acc


def _topk_call(xt, x, cdim, slot_base):
    return pl.pallas_call(
        functools.partial(_topk_body, cdim, slot_base),
        grid=(B, NBLK),
        in_specs=[
            pl.BlockSpec((1, TN, cdim), lambda b, n: (b, n, 0)),
            pl.BlockSpec((1, cdim, N), lambda b, n: (b, 0, 0)),
        ],
        out_specs=pl.BlockSpec((1, TN, 16), lambda b, n: (b, n, 0)),
        out_shape=jax.ShapeDtypeStruct((B, N, 16), jnp.int32),
    )(xt, x)


def _mm_body(xt_ref, w_ref, o_ref):
    o_ref[0, 0] = jnp.dot(xt_ref[0], w_ref[0],
                          preferred_element_type=jnp.float32)


def _transforms_call(xt, wstack):
    return pl.pallas_call(
        _mm_body,
        grid=(B, NSLOT),
        in_specs=[
            pl.BlockSpec((1, N, C), lambda b, s: (b, 0, 0)),
            pl.BlockSpec((1, C, C), lambda b, s: (s, 0, 0)),
        ],
        out_specs=pl.BlockSpec((1, 1, N, C), lambda b, s: (b, s, 0, 0)),
        out_shape=jax.ShapeDtypeStruct((B, NSLOT, N, C), jnp.float32),
    )(xt, wstack)


def _sc_gather_body(tab_hbm, idx_hbm, out_hbm, idx_v, buf_v, sem):
    wid = lax.axis_index("s") * 2 + lax.axis_index("c")
    base = wid * RPW
    pltpu.sync_copy(idx_hbm.at[pl.ds(base, RPW)], idx_v)

    def chunk(g, carry):
        off = g * CHUNK
        cp = pltpu.async_copy(
            tab_hbm.at[idx_v.at[pl.ds(off, CHUNK)]], buf_v, sem)
        cp.wait()
        pltpu.sync_copy(buf_v, out_hbm.at[pl.ds(base + off, CHUNK)])
        return carry

    lax.fori_loop(0, NCH, chunk, 0)


def _gather_rows(table, flat_idx):
    fn = pl.kernel(
        _sc_gather_body,
        mesh=plsc.VectorSubcoreMesh(core_axis_name="c", subcore_axis_name="s"),
        out_type=jax.ShapeDtypeStruct((TOT, C), jnp.float32),
        scratch_types=[
            pltpu.VMEM((RPW,), jnp.int32),
            pltpu.VMEM((CHUNK, C), jnp.float32),
            pltpu.SemaphoreType.DMA,
        ],
    )
    return fn(table, flat_idx)


def _assemble_body(g_ref, c_ref, a_ref, st_ref):
    first = jnp.logical_and(pl.program_id(1) == 0, pl.program_id(2) == 0)

    @pl.when(first)
    def _():
        st_ref[0] = jnp.zeros((8, C), jnp.float32)

    g = g_ref[0, 0]
    cen = c_ref[0, 0]
    ssum = jnp.zeros((1, C), jnp.float32)
    ssq = jnp.zeros((1, C), jnp.float32)
    for j in range(3):
        sj = (g[:, (3 * j) * C:(3 * j + 1) * C]
              + g[:, (3 * j + 1) * C:(3 * j + 2) * C]
              + g[:, (3 * j + 2) * C:(3 * j + 3) * C])
        aj = cen - sj
        a_ref[0, 0, :, j * C:(j + 1) * C] = aj
        ssum = ssum + jnp.sum(aj, axis=0, keepdims=True)
        ssq = ssq + jnp.sum(aj * aj, axis=0, keepdims=True)
    st_ref[0, 0:1, :] = st_ref[0, 0:1, :] + ssum
    st_ref[0, 1:2, :] = st_ref[0, 1:2, :] + ssq


def _assemble_call(g2d, stack):
    return pl.pallas_call(
        _assemble_body,
        grid=(2, B, NB),
        in_specs=[
            pl.BlockSpec((1, 1, TR, K * C), lambda br, b, n: (br, b, n, 0)),
            pl.BlockSpec((1, 1, TR, C), lambda br, b, n: (b, 4 * br, n, 0)),
        ],
        out_specs=[
            pl.BlockSpec((1, 1, TR, 3 * C), lambda br, b, n: (br, b, n, 0)),
            pl.BlockSpec((1, 8, C), lambda br, b, n: (br, 0, 0)),
        ],
        out_shape=[
            jax.ShapeDtypeStruct((2, B, N, 3 * C), jnp.float32),
            jax.ShapeDtypeStruct((2, 8, C), jnp.float32),
        ],
    )(g2d, stack)


def _conv2_body(a_ref, st_ref, gb_ref, w_ref, o_ref, s2_ref):
    first = jnp.logical_and(pl.program_id(1) == 0, pl.program_id(2) == 0)

    @pl.when(first)
    def _():
        s2_ref[0] = jnp.zeros((8, C), jnp.float32)

    mean = st_ref[0, 0:1, :] * (1.0 / CNT1)
    var = st_ref[0, 1:2, :] * (1.0 / CNT1) - mean * mean
    rstd = lax.rsqrt(var + EPS)
    scale = gb_ref[0, 0:1, :] * rstd
    shift = gb_ref[0, 1:2, :] - mean * scale
    a = a_ref[0, 0]
    acc = jnp.zeros((TR, C), jnp.float32)
    for j in range(3):
        h = jnp.maximum(a[:, j * C:(j + 1) * C] * scale + shift, 0.0)
        acc = acc + jnp.dot(h, w_ref[0, j],
                            preferred_element_type=jnp.float32)
    o_ref[0, 0] = acc
    s2_ref[0, 0:1, :] = s2_ref[0, 0:1, :] + jnp.sum(acc, 0, keepdims=True)
    s2_ref[0, 1:2, :] = s2_ref[0, 1:2, :] + jnp.sum(acc * acc, 0,
                                                    keepdims=True)


def _conv2_call(a4d, st1, gb1, w2t):
    return pl.pallas_call(
        _conv2_body,
        grid=(2, B, NB),
        in_specs=[
            pl.BlockSpec((1, 1, TR, 3 * C), lambda br, b, n: (br, b, n, 0)),
            pl.BlockSpec((1, 8, C), lambda br, b, n: (br, 0, 0)),
            pl.BlockSpec((1, 8, C), lambda br, b, n: (br, 0, 0)),
            pl.BlockSpec((1, 3, C, C), lambda br, b, n: (br, 0, 0, 0)),
        ],
        out_specs=[
            pl.BlockSpec((1, 1, TR, C), lambda br, b, n: (br, b, n, 0)),
            pl.BlockSpec((1, 8, C), lambda br, b, n: (br, 0, 0)),
        ],
        out_shape=[
            jax.ShapeDtypeStruct((2, B, N, C), jnp.float32),
            jax.ShapeDtypeStruct((2, 8, C), jnp.float32),
        ],
    )(a4d, st1, gb1, w2t)


def _final_body(o1_ref, o2_ref, s2_ref, gb_ref, out_ref):
    res = None
    for br in range(2):
        mean = s2_ref[br, 0:1, :] * (1.0 / CNT2)
        var = s2_ref[br, 1:2, :] * (1.0 / CNT2) - mean * mean
        rstd = lax.rsqrt(var + EPS)
        scale = gb_ref[br, 0:1, :] * rstd
        shift = gb_ref[br, 1:2, :] - mean * scale
        o = o1_ref[0] if br == 0 else o2_ref[0]
        y = jnp.maximum(o * scale + shift, 0.0)
        if br == 0:
            res = y
        else:
            res = res + gb_ref[1, 2:3, :] * y
    out_ref[0] = res


def _final_call(o1, o2, st2, gb2):
    return pl.pallas_call(
        _final_body,
        grid=(B, NB),
        in_specs=[
            pl.BlockSpec((1, TR, C), lambda b, n: (b, n, 0)),
            pl.BlockSpec((1, TR, C), lambda b, n: (b, n, 0)),
            pl.BlockSpec((2, 8, C), lambda b, n: (0, 0, 0)),
            pl.BlockSpec((2, 8, C), lambda b, n: (0, 0, 0)),
        ],
        out_specs=pl.BlockSpec((1, TR, C), lambda b, n: (b, n, 0)),
        out_shape=jax.ShapeDtypeStruct((B, N, C), jnp.float32),
    )(o1, o2, st2, gb2)


def kernel(features, motion, W1, b1, g1, be1, W2, b2, g2, be2,
           W3, b3, g3, be3, W4, b4, g4, be4, delta):
    x = features.reshape(B, C, N)
    xt = jnp.transpose(x, (0, 2, 1))
    m = motion.reshape(B, 2, N)
    m8 = jnp.concatenate([m, jnp.zeros((B, 6, N), jnp.float32)], axis=1)
    m8t = jnp.transpose(m8, (0, 2, 1))

    def slots(w):
        wa = [w[:, :C, 0, t] for t in range(3)]
        wb = [w[:, C:, 0, t] for t in range(3)]
        cen = sum(wa[t] + wb[t] for t in range(3)).T
        return [cen] + [wb[t].T for t in range(3)]

    wstack = jnp.stack(slots(W1) + slots(W3))
    w2t = jnp.stack([
        jnp.stack([W2[:, :, 0, t].T for t in range(3)]),
        jnp.stack([W4[:, :, 0, t].T for t in range(3)]),
    ])
    zrow = jnp.zeros((1, C), jnp.float32)
    drow = jnp.broadcast_to(delta.reshape(1, 1), (1, C))

    def gbrows(g, be, third):
        return jnp.concatenate(
            [g[None], be[None], third] + [zrow] * 5, axis=0)

    gb1 = jnp.stack([gbrows(g1, be1, zrow), gbrows(g3, be3, zrow)])
    gb2 = jnp.stack([gbrows(g2, be2, drow), gbrows(g4, be4, drow)])

    i1 = _topk_call(xt, x, C, 1)[:, :, :K]
    i2 = _topk_call(m8t, m8, 8, 5)[:, :, :K]
    flat_idx = jnp.stack([i1, i2]).reshape(-1)

    stack = _transforms_call(xt, wstack)
    table = stack.reshape(B * NSLOT * N, C)

    g_rows = _gather_rows(table, flat_idx)
    g2d = g_rows.reshape(2, B, N, K * C)

    a4d, st1 = _assemble_call(g2d, stack)
    o, st2 = _conv2_call(a4d, st1, gb1, w2t)
    res = _final_call(o[0], o[1], st2, gb2)
    return jnp.transpose(res, (0, 2, 1))[..., None]


# speedup vs baseline: 13.8512x; 1.0382x over previous
import functools

import jax
import jax.numpy as jnp
from jax import lax
from jax.experimental import pallas as pl
from jax.experimental.pallas import tpu as pltpu
from jax.experimental.pallas import tpu_sc as plsc

B, C, N = 8, 128, 2048
K = 9
TN = 256
NBLK = N // TN
TR = 512
NB = N // TR
NSLOT = 8
TOT = 2 * B * N * K
NW = 32
RPW = TOT // NW
CHUNK = 128
NCH = RPW // CHUNK
CNT1 = B * N * 3
CNT2 = B * N
EPS = 1e-5
NEG = -3.0e38
BIGI = 1 << 30


def _topk_body(cdim, slot_base, xt_ref, x_ref, o_ref):
    b = pl.program_id(0)
    xb = xt_ref[0]
    xf = x_ref[0]
    xx = jnp.sum(xf * xf, axis=0, keepdims=True)
    s = jnp.dot(xb, xf, preferred_element_type=jnp.float32)
    v = 2.0 * s - xx
    iota_m = lax.broadcasted_iota(jnp.int32, (TN, N), 1)
    iota16 = lax.broadcasted_iota(jnp.int32, (TN, 16), 1)
    acc = jnp.zeros((TN, 16), jnp.int32)
    gbase = b * (NSLOT * N)
    for r in range(K):
        mx = jnp.max(v, axis=1, keepdims=True)
        hit = v >= mx
        idxr = jnp.min(jnp.where(hit, iota_m, BIGI), axis=1, keepdims=True)
        slot = slot_base + (r % 3)
        gidx = idxr + (gbase + slot * N)
        acc = jnp.where(iota16 == r, jnp.broadcast_to(gidx, (TN, 16)), acc)
        v = jnp.where(iota_m == idxr, NEG, v)
    o_ref[0] = acc


def _topk_call(xt, x, cdim, slot_base):
    return pl.pallas_call(
        functools.partial(_topk_body, cdim, slot_base),
        grid=(B, NBLK),
        in_specs=[
            pl.BlockSpec((1, TN, cdim), lambda b, n: (b, n, 0)),
            pl.BlockSpec((1, cdim, N), lambda b, n: (b, 0, 0)),
        ],
        out_specs=pl.BlockSpec((1, TN, 16), lambda b, n: (b, n, 0)),
        out_shape=jax.ShapeDtypeStruct((B, N, 16), jnp.int32),
    )(xt, x)


def _mm_body(xt_ref, w_ref, o_ref):
    o_ref[0, 0] = jnp.dot(xt_ref[0], w_ref[0],
                          preferred_element_type=jnp.float32)


def _transforms_call(xt, wstack):
    return pl.pallas_call(
        _mm_body,
        grid=(B, NSLOT),
        in_specs=[
            pl.BlockSpec((1, N, C), lambda b, s: (b, 0, 0)),
            pl.BlockSpec((1, C, C), lambda b, s: (s, 0, 0)),
        ],
        out_specs=pl.BlockSpec((1, 1, N, C), lambda b, s: (b, s, 0, 0)),
        out_shape=jax.ShapeDtypeStruct((B, NSLOT, N, C), jnp.float32),
    )(xt, wstack)


def _sc_gather_body(tab_hbm, idx_hbm, out_hbm, idx_v, buf_v, sem0, sem1):
    wid = lax.axis_index("s") * 2 + lax.axis_index("c")
    base = wid * RPW
    sems = [sem0, sem1]
    pltpu.sync_copy(idx_hbm.at[pl.ds(base, RPW)], idx_v)
    for p in range(2):
        pltpu.async_copy(
            tab_hbm.at[idx_v.at[pl.ds(p * CHUNK, CHUNK)]],
            buf_v.at[p], sems[p])

    def outer(g0, carry):
        for p in range(2):
            g = g0 * 2 + p
            off = g * CHUNK
            pltpu.make_async_copy(
                tab_hbm.at[idx_v.at[pl.ds(off, CHUNK)]],
                buf_v.at[p], sems[p]).wait()
            pltpu.sync_copy(buf_v.at[p],
                            out_hbm.at[pl.ds(base + off, CHUNK)])

            @pl.when(g + 2 < NCH)
            def _():
                off2 = (g + 2) * CHUNK
                pltpu.async_copy(
                    tab_hbm.at[idx_v.at[pl.ds(off2, CHUNK)]],
                    buf_v.at[p], sems[p])
        return carry

    lax.fori_loop(0, NCH // 2, outer, 0)


def _gather_rows(table, flat_idx):
    fn = pl.kernel(
        _sc_gather_body,
        mesh=plsc.VectorSubcoreMesh(core_axis_name="c", subcore_axis_name="s"),
        out_type=jax.ShapeDtypeStruct((TOT, C), jnp.float32),
        scratch_types=[
            pltpu.VMEM((RPW,), jnp.int32),
            pltpu.VMEM((2, CHUNK, C), jnp.float32),
            pltpu.SemaphoreType.DMA,
            pltpu.SemaphoreType.DMA,
        ],
    )
    return fn(table, flat_idx)


def _assemble_body(g_ref, c_ref, a_ref, st_ref):
    first = jnp.logical_and(pl.program_id(1) == 0, pl.program_id(2) == 0)

    @pl.when(first)
    def _():
        st_ref[0] = jnp.zeros((8, C), jnp.float32)

    g = g_ref[0, 0]
    cen = c_ref[0, 0]
    ssum = jnp.zeros((1, C), jnp.float32)
    ssq = jnp.zeros((1, C), jnp.float32)
    for j in range(3):
        sj = (g[:, (3 * j) * C:(3 * j + 1) * C]
              + g[:, (3 * j + 1) * C:(3 * j + 2) * C]
              + g[:, (3 * j + 2) * C:(3 * j + 3) * C])
        aj = cen - sj
        a_ref[0, 0, :, j * C:(j + 1) * C] = aj
        ssum = ssum + jnp.sum(aj, axis=0, keepdims=True)
        ssq = ssq + jnp.sum(aj * aj, axis=0, keepdims=True)
    st_ref[0, 0:1, :] = st_ref[0, 0:1, :] + ssum
    st_ref[0, 1:2, :] = st_ref[0, 1:2, :] + ssq


def _assemble_call(g2d, stack):
    return pl.pallas_call(
        _assemble_body,
        grid=(2, B, NB),
        in_specs=[
            pl.BlockSpec((1, 1, TR, K * C), lambda br, b, n: (br, b, n, 0)),
            pl.BlockSpec((1, 1, TR, C), lambda br, b, n: (b, 4 * br, n, 0)),
        ],
        out_specs=[
            pl.BlockSpec((1, 1, TR, 3 * C), lambda br, b, n: (br, b, n, 0)),
            pl.BlockSpec((1, 8, C), lambda br, b, n: (br, 0, 0)),
        ],
        out_shape=[
            jax.ShapeDtypeStruct((2, B, N, 3 * C), jnp.float32),
            jax.ShapeDtypeStruct((2, 8, C), jnp.float32),
        ],
    )(g2d, stack)


def _conv2_body(a_ref, st_ref, gb_ref, w_ref, o_ref, s2_ref):
    first = jnp.logical_and(pl.program_id(1) == 0, pl.program_id(2) == 0)

    @pl.when(first)
    def _():
        s2_ref[0] = jnp.zeros((8, C), jnp.float32)

    mean = st_ref[0, 0:1, :] * (1.0 / CNT1)
    var = st_ref[0, 1:2, :] * (1.0 / CNT1) - mean * mean
    rstd = lax.rsqrt(var + EPS)
    scale = gb_ref[0, 0:1, :] * rstd
    shift = gb_ref[0, 1:2, :] - mean * scale
    a = a_ref[0, 0]
    acc = jnp.zeros((TR, C), jnp.float32)
    for j in range(3):
        h = jnp.maximum(a[:, j * C:(j + 1) * C] * scale + shift, 0.0)
        acc = acc + jnp.dot(h, w_ref[0, j],
                            preferred_element_type=jnp.float32)
    o_ref[0, 0] = acc
    s2_ref[0, 0:1, :] = s2_ref[0, 0:1, :] + jnp.sum(acc, 0, keepdims=True)
    s2_ref[0, 1:2, :] = s2_ref[0, 1:2, :] + jnp.sum(acc * acc, 0,
                                                    keepdims=True)


def _conv2_call(a4d, st1, gb1, w2t):
    return pl.pallas_call(
        _conv2_body,
        grid=(2, B, NB),
        in_specs=[
            pl.BlockSpec((1, 1, TR, 3 * C), lambda br, b, n: (br, b, n, 0)),
            pl.BlockSpec((1, 8, C), lambda br, b, n: (br, 0, 0)),
            pl.BlockSpec((1, 8, C), lambda br, b, n: (br, 0, 0)),
            pl.BlockSpec((1, 3, C, C), lambda br, b, n: (br, 0, 0, 0)),
        ],
        out_specs=[
            pl.BlockSpec((1, 1, TR, C), lambda br, b, n: (br, b, n, 0)),
            pl.BlockSpec((1, 8, C), lambda br, b, n: (br, 0, 0)),
        ],
        out_shape=[
            jax.ShapeDtypeStruct((2, B, N, C), jnp.float32),
            jax.ShapeDtypeStruct((2, 8, C), jnp.float32),
        ],
    )(a4d, st1, gb1, w2t)


def _final_body(o1_ref, o2_ref, s2_ref, gb_ref, out_ref):
    res = None
    for br in range(2):
        mean = s2_ref[br, 0:1, :] * (1.0 / CNT2)
        var = s2_ref[br, 1:2, :] * (1.0 / CNT2) - mean * mean
        rstd = lax.rsqrt(var + EPS)
        scale = gb_ref[br, 0:1, :] * rstd
        shift = gb_ref[br, 1:2, :] - mean * scale
        o = o1_ref[0] if br == 0 else o2_ref[0]
        y = jnp.maximum(o * scale + shift, 0.0)
        if br == 0:
            res = y
        else:
            res = res + gb_ref[1, 2:3, :] * y
    out_ref[0] = res


def _final_call(o1, o2, st2, gb2):
    return pl.pallas_call(
        _final_body,
        grid=(B, NB),
        in_specs=[
            pl.BlockSpec((1, TR, C), lambda b, n: (b, n, 0)),
            pl.BlockSpec((1, TR, C), lambda b, n: (b, n, 0)),
            pl.BlockSpec((2, 8, C), lambda b, n: (0, 0, 0)),
            pl.BlockSpec((2, 8, C), lambda b, n: (0, 0, 0)),
        ],
        out_specs=pl.BlockSpec((1, TR, C), lambda b, n: (b, n, 0)),
        out_shape=jax.ShapeDtypeStruct((B, N, C), jnp.float32),
    )(o1, o2, st2, gb2)


def kernel(features, motion, W1, b1, g1, be1, W2, b2, g2, be2,
           W3, b3, g3, be3, W4, b4, g4, be4, delta):
    x = features.reshape(B, C, N)
    xt = jnp.transpose(x, (0, 2, 1))
    m = motion.reshape(B, 2, N)
    m8 = jnp.concatenate([m, jnp.zeros((B, 6, N), jnp.float32)], axis=1)
    m8t = jnp.transpose(m8, (0, 2, 1))

    def slots(w):
        wa = [w[:, :C, 0, t] for t in range(3)]
        wb = [w[:, C:, 0, t] for t in range(3)]
        cen = sum(wa[t] + wb[t] for t in range(3)).T
        return [cen] + [wb[t].T for t in range(3)]

    wstack = jnp.stack(slots(W1) + slots(W3))
    w2t = jnp.stack([
        jnp.stack([W2[:, :, 0, t].T for t in range(3)]),
        jnp.stack([W4[:, :, 0, t].T for t in range(3)]),
    ])
    zrow = jnp.zeros((1, C), jnp.float32)
    drow = jnp.broadcast_to(delta.reshape(1, 1), (1, C))

    def gbrows(g, be, third):
        return jnp.concatenate(
            [g[None], be[None], third] + [zrow] * 5, axis=0)

    gb1 = jnp.stack([gbrows(g1, be1, zrow), gbrows(g3, be3, zrow)])
    gb2 = jnp.stack([gbrows(g2, be2, drow), gbrows(g4, be4, drow)])

    i1 = _topk_call(xt, x, C, 1)[:, :, :K]
    i2 = _topk_call(m8t, m8, 8, 5)[:, :, :K]
    flat_idx = jnp.stack([i1, i2]).reshape(-1)

    stack = _transforms_call(xt, wstack)
    table = stack.reshape(B * NSLOT * N, C)

    g_rows = _gather_rows(table, flat_idx)
    g2d = g_rows.reshape(2, B, N, K * C)

    a4d, st1 = _assemble_call(g2d, stack)
    o, st2 = _conv2_call(a4d, st1, gb1, w2t)
    res = _final_call(o[0], o[1], st2, gb2)
    return jnp.transpose(res, (0, 2, 1))[..., None]


# speedup vs baseline: 16.8676x; 1.2178x over previous
import functools

import jax
import jax.numpy as jnp
from jax import lax
from jax.experimental import pallas as pl
from jax.experimental.pallas import tpu as pltpu
from jax.experimental.pallas import tpu_sc as plsc

B, C, N = 8, 128, 2048
K = 9
TN = 256
NBLK = N // TN
TR = 512
NB = N // TR
NSLOT = 8
TOT = 2 * B * N * K
NW = 32
RPW = TOT // NW
CHUNK = 128
NCH = RPW // CHUNK
CNT1 = B * N * 3
CNT2 = B * N
EPS = 1e-5
NEG = -3.0e38
BIGI = 1 << 30


def _topk_body(cdim, slot_base, xt_ref, x_ref, o_ref):
    b = pl.program_id(0)
    xb = xt_ref[0]
    xf = x_ref[0]
    xx = jnp.sum(xf * xf, axis=0, keepdims=True)
    s = jnp.dot(xb, xf, preferred_element_type=jnp.float32)
    v = 2.0 * s - xx
    iota_m = lax.broadcasted_iota(jnp.int32, (TN, N), 1)
    iota_f = lax.broadcasted_iota(jnp.int32, (TN, N), 1).astype(jnp.float32)
    iota16 = lax.broadcasted_iota(jnp.int32, (TN, 16), 1)
    acc = jnp.zeros((TN, 16), jnp.int32)
    gbase = b * (NSLOT * N)
    for r in range(K):
        if r == 0:
            idxi = (lax.broadcasted_iota(jnp.int32, (TN, 1), 0)
                    + pl.program_id(1) * TN)
        else:
            mx = jnp.max(v, axis=1, keepdims=True)
            idxf = jnp.min(jnp.where(v >= mx, iota_f, 3.0e38),
                           axis=1, keepdims=True)
            idxi = idxf.astype(jnp.int32)
        slot = slot_base + (r % 3)
        gidx = idxi + (gbase + slot * N)
        acc = jnp.where(iota16 == r, jnp.broadcast_to(gidx, (TN, 16)), acc)
        v = jnp.where(iota_m == idxi, NEG, v)
    o_ref[0] = acc


def _topk_call(xt, x, cdim, slot_base):
    return pl.pallas_call(
        functools.partial(_topk_body, cdim, slot_base),
        grid=(B, NBLK),
        in_specs=[
            pl.BlockSpec((1, TN, cdim), lambda b, n: (b, n, 0)),
            pl.BlockSpec((1, cdim, N), lambda b, n: (b, 0, 0)),
        ],
        out_specs=pl.BlockSpec((1, TN, 16), lambda b, n: (b, n, 0)),
        out_shape=jax.ShapeDtypeStruct((B, N, 16), jnp.int32),
    )(xt, x)


def _mm_body(xt_ref, w_ref, o_ref):
    o_ref[0, 0] = jnp.dot(xt_ref[0], w_ref[0],
                          preferred_element_type=jnp.float32)


def _transforms_call(xt, wstack):
    return pl.pallas_call(
        _mm_body,
        grid=(B, NSLOT),
        in_specs=[
            pl.BlockSpec((1, N, C), lambda b, s: (b, 0, 0)),
            pl.BlockSpec((1, C, C), lambda b, s: (s, 0, 0)),
        ],
        out_specs=pl.BlockSpec((1, 1, N, C), lambda b, s: (b, s, 0, 0)),
        out_shape=jax.ShapeDtypeStruct((B, NSLOT, N, C), jnp.float32),
    )(xt, wstack)


def _sc_gather_body(tab_hbm, idx_hbm, out_hbm, idx_v, buf_v, sem0, sem1):
    wid = lax.axis_index("s") * 2 + lax.axis_index("c")
    base = wid * RPW
    sems = [sem0, sem1]
    pltpu.sync_copy(idx_hbm.at[pl.ds(base, RPW)], idx_v)
    for p in range(2):
        pltpu.async_copy(
            tab_hbm.at[idx_v.at[pl.ds(p * CHUNK, CHUNK)]],
            buf_v.at[p], sems[p])

    def outer(g0, carry):
        for p in range(2):
            g = g0 * 2 + p
            off = g * CHUNK
            pltpu.make_async_copy(
                tab_hbm.at[idx_v.at[pl.ds(off, CHUNK)]],
                buf_v.at[p], sems[p]).wait()
            pltpu.sync_copy(buf_v.at[p],
                            out_hbm.at[pl.ds(base + off, CHUNK)])

            @pl.when(g + 2 < NCH)
            def _():
                off2 = (g + 2) * CHUNK
                pltpu.async_copy(
                    tab_hbm.at[idx_v.at[pl.ds(off2, CHUNK)]],
                    buf_v.at[p], sems[p])
        return carry

    lax.fori_loop(0, NCH // 2, outer, 0)


def _gather_rows(table, flat_idx):
    fn = pl.kernel(
        _sc_gather_body,
        mesh=plsc.VectorSubcoreMesh(core_axis_name="c", subcore_axis_name="s"),
        out_type=jax.ShapeDtypeStruct((TOT, C), jnp.float32),
        scratch_types=[
            pltpu.VMEM((RPW,), jnp.int32),
            pltpu.VMEM((2, CHUNK, C), jnp.float32),
            pltpu.SemaphoreType.DMA,
            pltpu.SemaphoreType.DMA,
        ],
    )
    return fn(table, flat_idx)


def _assemble_body(g_ref, c_ref, a_ref, st_ref):
    first = jnp.logical_and(pl.program_id(1) == 0, pl.program_id(2) == 0)

    @pl.when(first)
    def _():
        st_ref[0] = jnp.zeros((8, C), jnp.float32)

    g = g_ref[0, 0]
    cen = c_ref[0, 0]
    ssum = jnp.zeros((1, C), jnp.float32)
    ssq = jnp.zeros((1, C), jnp.float32)
    for j in range(3):
        sj = (g[:, (3 * j) * C:(3 * j + 1) * C]
              + g[:, (3 * j + 1) * C:(3 * j + 2) * C]
              + g[:, (3 * j + 2) * C:(3 * j + 3) * C])
        aj = cen - sj
        a_ref[0, 0, :, j * C:(j + 1) * C] = aj
        ssum = ssum + jnp.sum(aj, axis=0, keepdims=True)
        ssq = ssq + jnp.sum(aj * aj, axis=0, keepdims=True)
    st_ref[0, 0:1, :] = st_ref[0, 0:1, :] + ssum
    st_ref[0, 1:2, :] = st_ref[0, 1:2, :] + ssq


def _assemble_call(g2d, stack):
    return pl.pallas_call(
        _assemble_body,
        grid=(2, B, NB),
        in_specs=[
            pl.BlockSpec((1, 1, TR, K * C), lambda br, b, n: (br, b, n, 0)),
            pl.BlockSpec((1, 1, TR, C), lambda br, b, n: (b, 4 * br, n, 0)),
        ],
        out_specs=[
            pl.BlockSpec((1, 1, TR, 3 * C), lambda br, b, n: (br, b, n, 0)),
            pl.BlockSpec((1, 8, C), lambda br, b, n: (br, 0, 0)),
        ],
        out_shape=[
            jax.ShapeDtypeStruct((2, B, N, 3 * C), jnp.float32),
            jax.ShapeDtypeStruct((2, 8, C), jnp.float32),
        ],
    )(g2d, stack)


def _conv2_body(a_ref, st_ref, gb_ref, w_ref, o_ref, s2_ref):
    first = jnp.logical_and(pl.program_id(1) == 0, pl.program_id(2) == 0)

    @pl.when(first)
    def _():
        s2_ref[0] = jnp.zeros((8, C), jnp.float32)

    mean = st_ref[0, 0:1, :] * (1.0 / CNT1)
    var = st_ref[0, 1:2, :] * (1.0 / CNT1) - mean * mean
    rstd = lax.rsqrt(var + EPS)
    scale = gb_ref[0, 0:1, :] * rstd
    shift = gb_ref[0, 1:2, :] - mean * scale
    a = a_ref[0, 0]
    acc = jnp.zeros((TR, C), jnp.float32)
    for j in range(3):
        h = jnp.maximum(a[:, j * C:(j + 1) * C] * scale + shift, 0.0)
        acc = acc + jnp.dot(h, w_ref[0, j],
                            preferred_element_type=jnp.float32)
    o_ref[0, 0] = acc
    s2_ref[0, 0:1, :] = s2_ref[0, 0:1, :] + jnp.sum(acc, 0, keepdims=True)
    s2_ref[0, 1:2, :] = s2_ref[0, 1:2, :] + jnp.sum(acc * acc, 0,
                                                    keepdims=True)


def _conv2_call(a4d, st1, gb1, w2t):
    return pl.pallas_call(
        _conv2_body,
        grid=(2, B, NB),
        in_specs=[
            pl.BlockSpec((1, 1, TR, 3 * C), lambda br, b, n: (br, b, n, 0)),
            pl.BlockSpec((1, 8, C), lambda br, b, n: (br, 0, 0)),
            pl.BlockSpec((1, 8, C), lambda br, b, n: (br, 0, 0)),
            pl.BlockSpec((1, 3, C, C), lambda br, b, n: (br, 0, 0, 0)),
        ],
        out_specs=[
            pl.BlockSpec((1, 1, TR, C), lambda br, b, n: (br, b, n, 0)),
            pl.BlockSpec((1, 8, C), lambda br, b, n: (br, 0, 0)),
        ],
        out_shape=[
            jax.ShapeDtypeStruct((2, B, N, C), jnp.float32),
            jax.ShapeDtypeStruct((2, 8, C), jnp.float32),
        ],
    )(a4d, st1, gb1, w2t)


def _final_body(o1_ref, o2_ref, s2_ref, gb_ref, out_ref):
    res = None
    for br in range(2):
        mean = s2_ref[br, 0:1, :] * (1.0 / CNT2)
        var = s2_ref[br, 1:2, :] * (1.0 / CNT2) - mean * mean
        rstd = lax.rsqrt(var + EPS)
        scale = gb_ref[br, 0:1, :] * rstd
        shift = gb_ref[br, 1:2, :] - mean * scale
        o = o1_ref[0] if br == 0 else o2_ref[0]
        y = jnp.maximum(o * scale + shift, 0.0)
        if br == 0:
            res = y
        else:
            res = res + gb_ref[1, 2:3, :] * y
    out_ref[0] = res


def _final_call(o1, o2, st2, gb2):
    return pl.pallas_call(
        _final_body,
        grid=(B, NB),
        in_specs=[
            pl.BlockSpec((1, TR, C), lambda b, n: (b, n, 0)),
            pl.BlockSpec((1, TR, C), lambda b, n: (b, n, 0)),
            pl.BlockSpec((2, 8, C), lambda b, n: (0, 0, 0)),
            pl.BlockSpec((2, 8, C), lambda b, n: (0, 0, 0)),
        ],
        out_specs=pl.BlockSpec((1, TR, C), lambda b, n: (b, n, 0)),
        out_shape=jax.ShapeDtypeStruct((B, N, C), jnp.float32),
    )(o1, o2, st2, gb2)


def kernel(features, motion, W1, b1, g1, be1, W2, b2, g2, be2,
           W3, b3, g3, be3, W4, b4, g4, be4, delta):
    x = features.reshape(B, C, N)
    xt = jnp.transpose(x, (0, 2, 1))
    m = motion.reshape(B, 2, N)
    m8 = jnp.concatenate([m, jnp.zeros((B, 6, N), jnp.float32)], axis=1)
    m8t = jnp.transpose(m8, (0, 2, 1))

    def slots(w):
        wa = [w[:, :C, 0, t] for t in range(3)]
        wb = [w[:, C:, 0, t] for t in range(3)]
        cen = sum(wa[t] + wb[t] for t in range(3)).T
        return [cen] + [wb[t].T for t in range(3)]

    wstack = jnp.stack(slots(W1) + slots(W3))
    w2t = jnp.stack([
        jnp.stack([W2[:, :, 0, t].T for t in range(3)]),
        jnp.stack([W4[:, :, 0, t].T for t in range(3)]),
    ])
    zrow = jnp.zeros((1, C), jnp.float32)
    drow = jnp.broadcast_to(delta.reshape(1, 1), (1, C))

    def gbrows(g, be, third):
        return jnp.concatenate(
            [g[None], be[None], third] + [zrow] * 5, axis=0)

    gb1 = jnp.stack([gbrows(g1, be1, zrow), gbrows(g3, be3, zrow)])
    gb2 = jnp.stack([gbrows(g2, be2, drow), gbrows(g4, be4, drow)])

    i1 = _topk_call(xt, x, C, 1)[:, :, :K]
    i2 = _topk_call(m8t, m8, 8, 5)[:, :, :K]
    flat_idx = jnp.stack([i1, i2]).reshape(-1)

    stack = _transforms_call(xt, wstack)
    table = stack.reshape(B * NSLOT * N, C)

    g_rows = _gather_rows(table, flat_idx)
    g2d = g_rows.reshape(2, B, N, K * C)

    a4d, st1 = _assemble_call(g2d, stack)
    o, st2 = _conv2_call(a4d, st1, gb1, w2t)
    res = _final_call(o[0], o[1], st2, gb2)
    return jnp.transpose(res, (0, 2, 1))[..., None]


# speedup vs baseline: 16.8845x; 1.0010x over previous
import functools

import jax
import jax.numpy as jnp
from jax import lax
from jax.experimental import pallas as pl
from jax.experimental.pallas import tpu as pltpu
from jax.experimental.pallas import tpu_sc as plsc

B, C, N = 8, 128, 2048
K = 9
TN = 256
NBLK = N // TN
TR = 512
NB = N // TR
NSLOT = 8
TOT = 2 * B * N * K
NW = 32
RPW = TOT // NW
CHUNK = 128
NCH = RPW // CHUNK
CNT1 = B * N * 3
CNT2 = B * N
EPS = 1e-5
NEG = -3.0e38
BIGI = 1 << 30


def _topk_body(cdim, slot_base, xt_ref, x_ref, o_ref):
    b = pl.program_id(0)
    xb = xt_ref[0]
    xf = x_ref[0]
    xx = jnp.sum(xf * xf, axis=0, keepdims=True)
    s = jnp.dot(xb, xf, preferred_element_type=jnp.float32)
    v = 2.0 * s - xx
    iota_m = lax.broadcasted_iota(jnp.int32, (TN, N), 1)
    iota_f = lax.broadcasted_iota(jnp.int32, (TN, N), 1).astype(jnp.float32)
    iota16 = lax.broadcasted_iota(jnp.int32, (TN, 16), 1)
    acc = jnp.zeros((TN, 16), jnp.int32)
    gbase = b * (NSLOT * N)
    nloc = pl.program_id(1) * TN
    rows16 = lax.broadcasted_iota(jnp.int32, (TN, 16), 0) + nloc
    acc = jnp.where(iota16 == 0, rows16 + (gbase + slot_base * N), acc)
    rows_m = lax.broadcasted_iota(jnp.int32, (TN, N), 0) + nloc
    v = jnp.where(iota_m == rows_m, NEG, v)
    for r in range(1, K):
        mx = jnp.max(v, axis=1, keepdims=True)
        idxf = jnp.min(jnp.where(v >= mx, iota_f, 3.0e38),
                       axis=1, keepdims=True)
        idxi = idxf.astype(jnp.int32)
        slot = slot_base + (r % 3)
        gidx = idxi + (gbase + slot * N)
        acc = jnp.where(iota16 == r, jnp.broadcast_to(gidx, (TN, 16)), acc)
        v = jnp.where(iota_m == idxi, NEG, v)
    o_ref[0] = acc


def _topk_call(xt, x, cdim, slot_base):
    return pl.pallas_call(
        functools.partial(_topk_body, cdim, slot_base),
        grid=(B, NBLK),
        in_specs=[
            pl.BlockSpec((1, TN, cdim), lambda b, n: (b, n, 0)),
            pl.BlockSpec((1, cdim, N), lambda b, n: (b, 0, 0)),
        ],
        out_specs=pl.BlockSpec((1, TN, 16), lambda b, n: (b, n, 0)),
        out_shape=jax.ShapeDtypeStruct((B, N, 16), jnp.int32),
    )(xt, x)


def _mm_body(xt_ref, w_ref, o_ref):
    o_ref[0, 0] = jnp.dot(xt_ref[0], w_ref[0],
                          preferred_element_type=jnp.float32)


def _transforms_call(xt, wstack):
    return pl.pallas_call(
        _mm_body,
        grid=(B, NSLOT),
        in_specs=[
            pl.BlockSpec((1, N, C), lambda b, s: (b, 0, 0)),
            pl.BlockSpec((1, C, C), lambda b, s: (s, 0, 0)),
        ],
        out_specs=pl.BlockSpec((1, 1, N, C), lambda b, s: (b, s, 0, 0)),
        out_shape=jax.ShapeDtypeStruct((B, NSLOT, N, C), jnp.float32),
    )(xt, wstack)


def _sc_gather_body(tab_hbm, idx_hbm, out_hbm, idx_v, buf_v, sem0, sem1):
    wid = lax.axis_index("s") * 2 + lax.axis_index("c")
    base = wid * RPW
    sems = [sem0, sem1]
    pltpu.sync_copy(idx_hbm.at[pl.ds(base, RPW)], idx_v)
    for p in range(2):
        pltpu.async_copy(
            tab_hbm.at[idx_v.at[pl.ds(p * CHUNK, CHUNK)]],
            buf_v.at[p], sems[p])

    def outer(g0, carry):
        for p in range(2):
            g = g0 * 2 + p
            off = g * CHUNK
            pltpu.make_async_copy(
                tab_hbm.at[idx_v.at[pl.ds(off, CHUNK)]],
                buf_v.at[p], sems[p]).wait()
            pltpu.sync_copy(buf_v.at[p],
                            out_hbm.at[pl.ds(base + off, CHUNK)])

            @pl.when(g + 2 < NCH)
            def _():
                off2 = (g + 2) * CHUNK
                pltpu.async_copy(
                    tab_hbm.at[idx_v.at[pl.ds(off2, CHUNK)]],
                    buf_v.at[p], sems[p])
        return carry

    lax.fori_loop(0, NCH // 2, outer, 0)


def _gather_rows(table, flat_idx):
    fn = pl.kernel(
        _sc_gather_body,
        mesh=plsc.VectorSubcoreMesh(core_axis_name="c", subcore_axis_name="s"),
        out_type=jax.ShapeDtypeStruct((TOT, C), jnp.float32),
        scratch_types=[
            pltpu.VMEM((RPW,), jnp.int32),
            pltpu.VMEM((2, CHUNK, C), jnp.float32),
            pltpu.SemaphoreType.DMA,
            pltpu.SemaphoreType.DMA,
        ],
    )
    return fn(table, flat_idx)


def _assemble_body(g_ref, c_ref, a_ref, st_ref):
    first = jnp.logical_and(pl.program_id(1) == 0, pl.program_id(2) == 0)

    @pl.when(first)
    def _():
        st_ref[0] = jnp.zeros((8, C), jnp.float32)

    g = g_ref[0, 0]
    cen = c_ref[0, 0]
    ssum = jnp.zeros((1, C), jnp.float32)
    ssq = jnp.zeros((1, C), jnp.float32)
    for j in range(3):
        sj = (g[:, (3 * j) * C:(3 * j + 1) * C]
              + g[:, (3 * j + 1) * C:(3 * j + 2) * C]
              + g[:, (3 * j + 2) * C:(3 * j + 3) * C])
        aj = cen - sj
        a_ref[0, 0, :, j * C:(j + 1) * C] = aj
        ssum = ssum + jnp.sum(aj, axis=0, keepdims=True)
        ssq = ssq + jnp.sum(aj * aj, axis=0, keepdims=True)
    st_ref[0, 0:1, :] = st_ref[0, 0:1, :] + ssum
    st_ref[0, 1:2, :] = st_ref[0, 1:2, :] + ssq


def _assemble_call(g2d, stack):
    return pl.pallas_call(
        _assemble_body,
        grid=(2, B, NB),
        in_specs=[
            pl.BlockSpec((1, 1, TR, K * C), lambda br, b, n: (br, b, n, 0)),
            pl.BlockSpec((1, 1, TR, C), lambda br, b, n: (b, 4 * br, n, 0)),
        ],
        out_specs=[
            pl.BlockSpec((1, 1, TR, 3 * C), lambda br, b, n: (br, b, n, 0)),
            pl.BlockSpec((1, 8, C), lambda br, b, n: (br, 0, 0)),
        ],
        out_shape=[
            jax.ShapeDtypeStruct((2, B, N, 3 * C), jnp.float32),
            jax.ShapeDtypeStruct((2, 8, C), jnp.float32),
        ],
    )(g2d, stack)


def _conv2_body(a_ref, st_ref, gb_ref, w_ref, o_ref, s2_ref):
    first = jnp.logical_and(pl.program_id(1) == 0, pl.program_id(2) == 0)

    @pl.when(first)
    def _():
        s2_ref[0] = jnp.zeros((8, C), jnp.float32)

    mean = st_ref[0, 0:1, :] * (1.0 / CNT1)
    var = st_ref[0, 1:2, :] * (1.0 / CNT1) - mean * mean
    rstd = lax.rsqrt(var + EPS)
    scale = gb_ref[0, 0:1, :] * rstd
    shift = gb_ref[0, 1:2, :] - mean * scale
    a = a_ref[0, 0]
    acc = jnp.zeros((TR, C), jnp.float32)
    for j in range(3):
        h = jnp.maximum(a[:, j * C:(j + 1) * C] * scale + shift, 0.0)
        acc = acc + jnp.dot(h, w_ref[0, j],
                            preferred_element_type=jnp.float32)
    o_ref[0, 0] = acc
    s2_ref[0, 0:1, :] = s2_ref[0, 0:1, :] + jnp.sum(acc, 0, keepdims=True)
    s2_ref[0, 1:2, :] = s2_ref[0, 1:2, :] + jnp.sum(acc * acc, 0,
                                                    keepdims=True)


def _conv2_call(a4d, st1, gb1, w2t):
    return pl.pallas_call(
        _conv2_body,
        grid=(2, B, NB),
        in_specs=[
            pl.BlockSpec((1, 1, TR, 3 * C), lambda br, b, n: (br, b, n, 0)),
            pl.BlockSpec((1, 8, C), lambda br, b, n: (br, 0, 0)),
            pl.BlockSpec((1, 8, C), lambda br, b, n: (br, 0, 0)),
            pl.BlockSpec((1, 3, C, C), lambda br, b, n: (br, 0, 0, 0)),
        ],
        out_specs=[
            pl.BlockSpec((1, 1, TR, C), lambda br, b, n: (br, b, n, 0)),
            pl.BlockSpec((1, 8, C), lambda br, b, n: (br, 0, 0)),
        ],
        out_shape=[
            jax.ShapeDtypeStruct((2, B, N, C), jnp.float32),
            jax.ShapeDtypeStruct((2, 8, C), jnp.float32),
        ],
    )(a4d, st1, gb1, w2t)


def _final_body(o1_ref, o2_ref, s2_ref, gb_ref, out_ref):
    res = None
    for br in range(2):
        mean = s2_ref[br, 0:1, :] * (1.0 / CNT2)
        var = s2_ref[br, 1:2, :] * (1.0 / CNT2) - mean * mean
        rstd = lax.rsqrt(var + EPS)
        scale = gb_ref[br, 0:1, :] * rstd
        shift = gb_ref[br, 1:2, :] - mean * scale
        o = o1_ref[0] if br == 0 else o2_ref[0]
        y = jnp.maximum(o * scale + shift, 0.0)
        if br == 0:
            res = y
        else:
            res = res + gb_ref[1, 2:3, :] * y
    out_ref[0] = res


def _final_call(o1, o2, st2, gb2):
    return pl.pallas_call(
        _final_body,
        grid=(B, NB),
        in_specs=[
            pl.BlockSpec((1, TR, C), lambda b, n: (b, n, 0)),
            pl.BlockSpec((1, TR, C), lambda b, n: (b, n, 0)),
            pl.BlockSpec((2, 8, C), lambda b, n: (0, 0, 0)),
            pl.BlockSpec((2, 8, C), lambda b, n: (0, 0, 0)),
        ],
        out_specs=pl.BlockSpec((1, TR, C), lambda b, n: (b, n, 0)),
        out_shape=jax.ShapeDtypeStruct((B, N, C), jnp.float32),
    )(o1, o2, st2, gb2)


def kernel(features, motion, W1, b1, g1, be1, W2, b2, g2, be2,
           W3, b3, g3, be3, W4, b4, g4, be4, delta):
    x = features.reshape(B, C, N)
    xt = jnp.transpose(x, (0, 2, 1))
    m = motion.reshape(B, 2, N)
    m8 = jnp.concatenate([m, jnp.zeros((B, 6, N), jnp.float32)], axis=1)
    m8t = jnp.transpose(m8, (0, 2, 1))

    def slots(w):
        wa = [w[:, :C, 0, t] for t in range(3)]
        wb = [w[:, C:, 0, t] for t in range(3)]
        cen = sum(wa[t] + wb[t] for t in range(3)).T
        return [cen] + [wb[t].T for t in range(3)]

    wstack = jnp.stack(slots(W1) + slots(W3))
    w2t = jnp.stack([
        jnp.stack([W2[:, :, 0, t].T for t in range(3)]),
        jnp.stack([W4[:, :, 0, t].T for t in range(3)]),
    ])
    zrow = jnp.zeros((1, C), jnp.float32)
    drow = jnp.broadcast_to(delta.reshape(1, 1), (1, C))

    def gbrows(g, be, third):
        return jnp.concatenate(
            [g[None], be[None], third] + [zrow] * 5, axis=0)

    gb1 = jnp.stack([gbrows(g1, be1, zrow), gbrows(g3, be3, zrow)])
    gb2 = jnp.stack([gbrows(g2, be2, drow), gbrows(g4, be4, drow)])

    i1 = _topk_call(xt, x, C, 1)[:, :, :K]
    i2 = _topk_call(m8t, m8, 8, 5)[:, :, :K]
    flat_idx = jnp.stack([i1, i2]).reshape(-1)

    stack = _transforms_call(xt, wstack)
    table = stack.reshape(B * NSLOT * N, C)

    g_rows = _gather_rows(table, flat_idx)
    g2d = g_rows.reshape(2, B, N, K * C)

    a4d, st1 = _assemble_call(g2d, stack)
    o, st2 = _conv2_call(a4d, st1, gb1, w2t)
    res = _final_call(o[0], o[1], st2, gb2)
    return jnp.transpose(res, (0, 2, 1))[..., None]
